# Initial kernel scaffold; baseline (speedup 1.0000x reference)
import jax
import jax.numpy as jnp
from jax.experimental import pallas as pl


def kernel(x_a, edge_index_a, batch_a, x_b, edge_index_b, batch_b, additional_features, W1, b1, g1, be1, W2, b2, g2, be2, W3, b3, g3, be3, M1, mb1, mg1, mbe1, M2, mb2, mg2, mbe2, M3, mb3):
    raise NotImplementedError("write your pallas kernel here")


# speedup vs baseline: 10.8829x; 10.8829x over previous
import functools

import jax
import jax.numpy as jnp
from jax import lax
from jax.experimental import pallas as pl
from jax.experimental.pallas import tpu as pltpu
from jax.experimental.pallas import tpu_sc as plsc

_EPS = 1e-5
_NS = 16
_CH = 128
_NEG = -3.0e38


def _ceil_div(a, b):
    return (a + b - 1) // b


def _fill(ref, n, value, dtype):
    for i in range(n // 16):
        ref[pl.ds(i * 16, 16)] = jnp.full((16,), value, dtype)


@functools.lru_cache(maxsize=None)
def _make_deg_cnt(N, E, G):
    ech = E // _CH
    e_tail = E % _CH
    assert e_tail == 0
    nch = N // _CH
    n_tail = N % _CH
    gch = _ceil_div(G, _CH)
    assert G % _CH == 0 or G < _CH
    mesh = plsc.VectorSubcoreMesh(core_axis_name="c", subcore_axis_name="s",
                                   num_cores=2, num_subcores=_NS)

    @functools.partial(
        pl.kernel,
        out_type=(
            jax.ShapeDtypeStruct((2 * N,), jnp.float32),
            jax.ShapeDtypeStruct((2 * G,), jnp.float32),
        ),
        mesh=mesh,
        scratch_types=[
            pltpu.VMEM((_CH,), jnp.int32),
            pltpu.VMEM((max(n_tail, 1),), jnp.int32),
            pltpu.VMEM((_CH,), jnp.float32),
            pltpu.VMEM((_CH,), jnp.float32),
            pltpu.VMEM_SHARED((N,), jnp.float32),
            pltpu.VMEM_SHARED((G,), jnp.float32),
        ],
    )
    def deg_cnt(dst_hbm, batch_hbm, degp, cntp, idx_v, tidx_v, ones_v, buf_v,
                deg_sh, cnt_sh):
        c = lax.axis_index("c")
        s = lax.axis_index("s")
        _fill(ones_v, _CH, 1.0, jnp.float32)
        _fill(buf_v, _CH, 0.0, jnp.float32)

        def zbody(j, carry):
            k = s + _NS * j

            @pl.when(k < nch)
            def _():
                pltpu.sync_copy(buf_v, deg_sh.at[pl.ds(k * _CH, _CH)])

            return carry

        lax.fori_loop(0, _ceil_div(nch, _NS), zbody, 0)
        if n_tail:
            @pl.when(s == nch % _NS)
            def _():
                pltpu.sync_copy(buf_v.at[pl.ds(0, n_tail)],
                                deg_sh.at[pl.ds(nch * _CH, n_tail)])
        for k in range(gch):
            @pl.when(s == k)
            def _():
                sz = min(_CH, G - k * _CH)
                pltpu.sync_copy(buf_v.at[pl.ds(0, sz)],
                                cnt_sh.at[pl.ds(k * _CH, sz)])
        plsc.subcore_barrier()

        def ebody(j, carry):
            k = s + _NS * j

            @pl.when(k < ech)
            def _():
                pltpu.sync_copy(dst_hbm.at[pl.ds(c * ech * _CH + k * _CH, _CH)],
                                idx_v)
                pltpu.sync_copy(ones_v, deg_sh.at[idx_v], add=True)

            return carry

        lax.fori_loop(0, _ceil_div(ech, _NS), ebody, 0)

        def bbody(j, carry):
            k = s + _NS * j

            @pl.when(k < nch)
            def _():
                pltpu.sync_copy(batch_hbm.at[pl.ds(c * N + k * _CH, _CH)],
                                idx_v)
                pltpu.sync_copy(ones_v, cnt_sh.at[idx_v], add=True)

            return carry

        lax.fori_loop(0, _ceil_div(nch, _NS), bbody, 0)
        if n_tail:
            @pl.when(s == nch % _NS)
            def _():
                pltpu.sync_copy(
                    batch_hbm.at[pl.ds(c * N + nch * _CH, n_tail)], tidx_v)
                pltpu.sync_copy(ones_v.at[pl.ds(0, n_tail)],
                                cnt_sh.at[tidx_v], add=True)
        plsc.subcore_barrier()

        def obody(j, carry):
            k = s + _NS * j

            @pl.when(k < nch)
            def _():
                pltpu.sync_copy(deg_sh.at[pl.ds(k * _CH, _CH)], buf_v)
                pltpu.sync_copy(buf_v,
                                degp.at[pl.ds(c * N + k * _CH, _CH)])

            return carry

        lax.fori_loop(0, _ceil_div(nch, _NS), obody, 0)
        if n_tail:
            @pl.when(s == nch % _NS)
            def _():
                pltpu.sync_copy(deg_sh.at[pl.ds(nch * _CH, n_tail)],
                                buf_v.at[pl.ds(0, n_tail)])
                pltpu.sync_copy(
                    buf_v.at[pl.ds(0, n_tail)],
                    degp.at[pl.ds(c * N + nch * _CH, n_tail)])
        for k in range(gch):
            @pl.when(s == _NS - 1 - k)
            def _():
                sz = min(_CH, G - k * _CH)
                pltpu.sync_copy(cnt_sh.at[pl.ds(k * _CH, sz)],
                                buf_v.at[pl.ds(0, sz)])
                pltpu.sync_copy(
                    buf_v.at[pl.ds(0, sz)],
                    cntp.at[pl.ds(c * G + k * _CH, sz)])

    return deg_cnt


@functools.lru_cache(maxsize=None)
def _make_msgpass(N, E, F):
    ech = E // _CH
    assert E % _CH == 0
    nch = N // _CH
    n_tail = N % _CH
    mesh = plsc.VectorSubcoreMesh(core_axis_name="c", subcore_axis_name="s",
                                   num_cores=2, num_subcores=_NS)

    @functools.partial(
        pl.kernel,
        out_type=jax.ShapeDtypeStruct((2, N, F), jnp.float32),
        mesh=mesh,
        scratch_types=[
            pltpu.VMEM((_CH,), jnp.int32),
            pltpu.VMEM((_CH,), jnp.int32),
            pltpu.VMEM((_CH, F), jnp.float32),
            pltpu.VMEM((_CH, F), jnp.float32),
            pltpu.VMEM_SHARED((N, F), jnp.float32),
            pltpu.SemaphoreType.DMA,
        ],
    )
    def msgpass(y_hbm, src_hbm, dst_hbm, accp, sidx_v, didx_v, rows_v,
                zrows_v, acc_sh, gsem):
        c = lax.axis_index("c")
        s = lax.axis_index("s")

        def zrow(i, carry):
            for v in range(F // 16):
                zrows_v[i, pl.ds(v * 16, 16)] = jnp.zeros((16,), jnp.float32)
            return carry

        lax.fori_loop(0, _CH, zrow, 0)

        def zbody(j, carry):
            k = s + _NS * j

            @pl.when(k < nch)
            def _():
                pltpu.sync_copy(zrows_v, acc_sh.at[pl.ds(k * _CH, _CH)])

            return carry

        lax.fori_loop(0, _ceil_div(nch, _NS), zbody, 0)
        if n_tail:
            @pl.when(s == nch % _NS)
            def _():
                pltpu.sync_copy(zrows_v.at[pl.ds(0, n_tail)],
                                acc_sh.at[pl.ds(nch * _CH, n_tail)])
        plsc.subcore_barrier()

        def ebody(j, carry):
            k = s + _NS * j

            @pl.when(k < ech)
            def _():
                pltpu.sync_copy(src_hbm.at[pl.ds(c * ech * _CH + k * _CH, _CH)],
                                sidx_v)
                pltpu.sync_copy(dst_hbm.at[pl.ds(c * ech * _CH + k * _CH, _CH)],
                                didx_v)
                pltpu.async_copy(y_hbm.at[sidx_v], rows_v, gsem).wait()
                pltpu.sync_copy(rows_v, acc_sh.at[didx_v], add=True)

            return carry

        lax.fori_loop(0, _ceil_div(ech, _NS), ebody, 0)
        plsc.subcore_barrier()

        def obody(j, carry):
            k = s + _NS * j

            @pl.when(k < nch)
            def _():
                pltpu.sync_copy(acc_sh.at[pl.ds(k * _CH, _CH)], rows_v)
                pltpu.sync_copy(rows_v, accp.at[c, pl.ds(k * _CH, _CH)])

            return carry

        lax.fori_loop(0, _ceil_div(nch, _NS), obody, 0)
        if n_tail:
            @pl.when(s == nch % _NS)
            def _():
                pltpu.sync_copy(acc_sh.at[pl.ds(nch * _CH, n_tail)],
                                rows_v.at[pl.ds(0, n_tail)])
                pltpu.sync_copy(rows_v.at[pl.ds(0, n_tail)],
                                accp.at[c, pl.ds(nch * _CH, n_tail)])

    return msgpass


@functools.lru_cache(maxsize=None)
def _make_pool(N, G, F):
    nch = N // _CH
    n_tail = N % _CH
    gch = _ceil_div(G, _CH)
    grows = G // _NS
    assert G % _NS == 0
    mesh = plsc.VectorSubcoreMesh(core_axis_name="c", subcore_axis_name="s",
                                   num_cores=2, num_subcores=_NS)

    @functools.partial(
        pl.kernel,
        out_type=(
            jax.ShapeDtypeStruct((2, G, F), jnp.float32),
            jax.ShapeDtypeStruct((2, G, F), jnp.float32),
        ),
        mesh=mesh,
        scratch_types=[
            pltpu.VMEM((_CH,), jnp.int32),
            pltpu.VMEM((max(n_tail, 1),), jnp.int32),
            pltpu.VMEM((_CH, F), jnp.float32),
            pltpu.VMEM((G, F), jnp.float32),
            pltpu.VMEM((_NS, grows, F), jnp.float32),
            pltpu.VMEM_SHARED((G, F), jnp.float32),
            pltpu.VMEM_SHARED((_NS, G, F), jnp.float32),
            pltpu.SemaphoreType.DMA,
        ],
    )
    def pool(x_hbm, batch_hbm, sump, maxp, bidx_v, tbidx_v, rows_v, macc_v,
             red_v, sum_sh, stage_sh, sem):
        c = lax.axis_index("c")
        s = lax.axis_index("s")

        def ibody(i, carry):
            for v in range(F // 16):
                macc_v[i, pl.ds(v * 16, 16)] = jnp.full((16,), _NEG,
                                                        jnp.float32)
            return carry

        lax.fori_loop(0, G, ibody, 0)

        def zrow(i, carry):
            for v in range(F // 16):
                rows_v[i, pl.ds(v * 16, 16)] = jnp.zeros((16,), jnp.float32)
            return carry

        lax.fori_loop(0, _CH, zrow, 0)
        for k in range(gch):
            @pl.when(s == k)
            def _():
                sz = min(_CH, G - k * _CH)
                pltpu.sync_copy(rows_v.at[pl.ds(0, sz)],
                                sum_sh.at[pl.ds(k * _CH, sz)])
        plsc.subcore_barrier()

        def process_chunk(k, nrows, bidx, tail):
            row_off = c * N + k * _CH
            pltpu.sync_copy(x_hbm.at[pl.ds(row_off, nrows)],
                            rows_v.at[pl.ds(0, nrows)] if tail else rows_v)
            pltpu.sync_copy(batch_hbm.at[pl.ds(c * N + k * _CH, nrows)],
                            bidx)
            pltpu.sync_copy(rows_v.at[pl.ds(0, nrows)] if tail else rows_v,
                            sum_sh.at[bidx], add=True)
            def gbody(g2, carry):
                bvec = bidx[pl.ds(g2 * 16, 16)]
                for ln in range(16):
                    b = bvec[ln]
                    row = g2 * 16 + ln
                    for v in range(F // 16):
                        sl = pl.ds(v * 16, 16)
                        macc_v[b, sl] = jnp.maximum(macc_v[b, sl],
                                                    rows_v[row, sl])
                return carry

            lax.fori_loop(0, nrows // 16, gbody, 0)

        def nbody(j, carry):
            k = s + _NS * j

            @pl.when(k < nch)
            def _():
                process_chunk(k, _CH, bidx_v, False)

            return carry

        lax.fori_loop(0, _ceil_div(nch, _NS), nbody, 0)
        if n_tail:
            @pl.when(s == nch % _NS)
            def _():
                process_chunk(nch, n_tail, tbidx_v, True)

        pltpu.sync_copy(macc_v, stage_sh.at[s])
        plsc.subcore_barrier()
        for t in range(_NS):
            pltpu.sync_copy(stage_sh.at[t, pl.ds(s * grows, grows)],
                            red_v.at[t])

        def rbody(r, carry):
            for v in range(F // 16):
                sl = pl.ds(v * 16, 16)
                m = red_v[0, r, sl]
                for t in range(1, _NS):
                    m = jnp.maximum(m, red_v[t, r, sl])
                red_v[0, r, sl] = m
            return carry

        lax.fori_loop(0, grows, rbody, 0)
        pltpu.sync_copy(red_v.at[0], maxp.at[c, pl.ds(s * grows, grows)])

        for k in range(gch):
            @pl.when(s == k)
            def _():
                sz = min(_CH, G - k * _CH)
                pltpu.sync_copy(sum_sh.at[pl.ds(k * _CH, sz)],
                                rows_v.at[pl.ds(0, sz)])
                pltpu.sync_copy(rows_v.at[pl.ds(0, sz)],
                                sump.at[c, pl.ds(k * _CH, sz)])

    return pool


def _dot(a, b):
    return jnp.dot(a, b, preferred_element_type=jnp.float32,
                   precision=lax.Precision.HIGHEST)


def _blk(N):
    for cand in (2000, 1250, 1000, 625, 500, 250, 125, 100, 80, 50, 40, 25,
                 20, 16, 10, 8, 5, 4, 2, 1):
        if N % cand == 0:
            return cand
    return 1


@functools.lru_cache(maxsize=None)
def _make_tc_pre(N, F, H):
    blk = _blk(N)
    nblk = 2 * N // blk

    def body(x_ref, degp_ref, w_ref, y_ref, dinv_ref):
        dinv = lax.rsqrt(degp_ref[...] + 1.0)
        y_ref[...] = _dot(x_ref[...], w_ref[...]) * dinv
        dinv_ref[...] = dinv

    return pl.pallas_call(
        body,
        grid=(nblk,),
        in_specs=[
            pl.BlockSpec((blk, F), lambda i: (i, 0)),
            pl.BlockSpec((blk, 1), lambda i: (i, 0)),
            pl.BlockSpec((F, H), lambda i: (0, 0)),
        ],
        out_specs=(
            pl.BlockSpec((blk, H), lambda i: (i, 0)),
            pl.BlockSpec((blk, 1), lambda i: (i, 0)),
        ),
        out_shape=(
            jax.ShapeDtypeStruct((2 * N, H), jnp.float32),
            jax.ShapeDtypeStruct((2 * N, 1), jnp.float32),
        ),
    )


@functools.lru_cache(maxsize=None)
def _make_tc_stats(N, H):
    blk = _blk(N)
    nblk = 2 * N // blk
    per_half = N // blk

    def body(acc_ref, y_ref, dinv_ref, b_ref, z_ref, s1_ref, s2_ref):
        i = pl.program_id(0)
        z = dinv_ref[...] * (acc_ref[...] + y_ref[...]) + b_ref[...]
        z_ref[...] = z
        s1 = jnp.sum(z, axis=0, keepdims=True).reshape(1, 1, -1)
        s2 = jnp.sum(z * z, axis=0, keepdims=True).reshape(1, 1, -1)

        @pl.when(i % per_half == 0)
        def _():
            s1_ref[...] = s1
            s2_ref[...] = s2

        @pl.when(i % per_half != 0)
        def _():
            s1_ref[...] += s1
            s2_ref[...] += s2

    return pl.pallas_call(
        body,
        grid=(nblk,),
        in_specs=[
            pl.BlockSpec((blk, H), lambda i: (i, 0)),
            pl.BlockSpec((blk, H), lambda i: (i, 0)),
            pl.BlockSpec((blk, 1), lambda i: (i, 0)),
            pl.BlockSpec((1, H), lambda i: (0, 0)),
        ],
        out_specs=(
            pl.BlockSpec((blk, H), lambda i: (i, 0)),
            pl.BlockSpec((1, 1, H), lambda i: (i // per_half, 0, 0)),
            pl.BlockSpec((1, 1, H), lambda i: (i // per_half, 0, 0)),
        ),
        out_shape=(
            jax.ShapeDtypeStruct((2 * N, H), jnp.float32),
            jax.ShapeDtypeStruct((2, 1, H), jnp.float32),
            jax.ShapeDtypeStruct((2, 1, H), jnp.float32),
        ),
    )


@functools.lru_cache(maxsize=None)
def _make_tc_apply(N, H, relu, matmul):
    blk = _blk(N)
    nblk = 2 * N // blk
    per_half = N // blk
    fN = float(N)

    def body(*refs):
        if matmul:
            (z_ref, s1_ref, s2_ref, g_ref, be_ref, w_ref, dinv_ref,
             o_ref) = refs
        else:
            z_ref, s1_ref, s2_ref, g_ref, be_ref, o_ref = refs
        m = s1_ref[...].reshape(1, -1) / fN
        v = s2_ref[...].reshape(1, -1) / fN - m * m
        z = (z_ref[...] - m) * lax.rsqrt(v + _EPS) * g_ref[...] + be_ref[...]
        if relu:
            z = jnp.maximum(z, 0.0)
        if matmul:
            z = _dot(z, w_ref[...]) * dinv_ref[...]
        o_ref[...] = z

    in_specs = [
        pl.BlockSpec((blk, H), lambda i: (i, 0)),
        pl.BlockSpec((1, 1, H), lambda i: (i // per_half, 0, 0)),
        pl.BlockSpec((1, 1, H), lambda i: (i // per_half, 0, 0)),
        pl.BlockSpec((1, H), lambda i: (0, 0)),
        pl.BlockSpec((1, H), lambda i: (0, 0)),
    ]
    if matmul:
        in_specs += [
            pl.BlockSpec((H, H), lambda i: (0, 0)),
            pl.BlockSpec((blk, 1), lambda i: (i, 0)),
        ]
    return pl.pallas_call(
        body,
        grid=(nblk,),
        in_specs=in_specs,
        out_specs=pl.BlockSpec((blk, H), lambda i: (i, 0)),
        out_shape=jax.ShapeDtypeStruct((2 * N, H), jnp.float32),
    )


def _layer(N, H, acc, y, dinv, b, g, be, W, relu, matmul):
    z, s1, s2 = _make_tc_stats(N, H)(acc.reshape(2 * N, H), y, dinv,
                                     b.reshape(1, H))
    args = (z, s1, s2, g.reshape(1, H), be.reshape(1, H))
    if matmul:
        args += (W, dinv)
    return _make_tc_apply(N, H, relu, matmul)(*args)


@functools.lru_cache(maxsize=None)
def _make_tc_head(G, H, ADD):
    CD = 4 * H + ADD

    def body(sump_ref, maxp_ref, cntp_ref, add_ref, m1_ref, mb1_ref, mg1_ref,
             mbe1_ref, m2_ref, mb2_ref, mg2_ref, mbe2_ref, m3_ref, mb3_ref,
             o_ref):
        cnt = jnp.maximum(cntp_ref[...].reshape(2, G), 1.0)
        xm = sump_ref[...] / cnt[:, :, None]
        xmx = maxp_ref[...]
        cat = jnp.concatenate(
            [xm[0], xmx[0], xm[1], xmx[1], add_ref[...]], axis=1)
        h = _dot(cat, m1_ref[...]) + mb1_ref[...]
        m = jnp.mean(h, axis=0, keepdims=True)
        v = jnp.mean(jnp.square(h - m), axis=0, keepdims=True)
        h = (h - m) * lax.rsqrt(v + _EPS) * mg1_ref[...] + mbe1_ref[...]
        h = jnp.maximum(h, 0.0)
        h2 = _dot(h, m2_ref[...]) + mb2_ref[...]
        m = jnp.mean(h2, axis=0, keepdims=True)
        v = jnp.mean(jnp.square(h2 - m), axis=0, keepdims=True)
        h2 = (h2 - m) * lax.rsqrt(v + _EPS) * mg2_ref[...] + mbe2_ref[...]
        h2 = jnp.maximum(h2, 0.0)
        logit = _dot(h2, m3_ref[...]) + mb3_ref[...]
        o_ref[...] = jax.nn.sigmoid(logit)[:, 0]

    return pl.pallas_call(
        body,
        out_shape=jax.ShapeDtypeStruct((G,), jnp.float32),
    )


def kernel(x_a, edge_index_a, batch_a, x_b, edge_index_b, batch_b,
           additional_features, W1, b1, g1, be1, W2, b2, g2, be2, W3, b3, g3,
           be3, M1, mb1, mg1, mbe1, M2, mb2, mg2, mbe2, M3, mb3):
    N, F_IN = x_a.shape
    E = edge_index_a.shape[1]
    G, ADD = additional_features.shape
    H = W1.shape[1]

    x = jnp.concatenate([x_a, x_b], axis=0)
    src = jnp.concatenate([edge_index_a[0], edge_index_b[0] + N])
    dst = jnp.concatenate([edge_index_a[1], edge_index_b[1]])
    batch = jnp.concatenate([batch_a, batch_b])

    degp, cntp = _make_deg_cnt(N, E, G)(dst, batch)

    y1, dinv = _make_tc_pre(N, F_IN, H)(x, degp.reshape(2 * N, 1), W1)
    acc1 = _make_msgpass(N, E, H)(y1, src, dst)
    y2 = _layer(N, H, acc1, y1, dinv, b1, g1, be1, W2, True, True)
    acc2 = _make_msgpass(N, E, H)(y2, src, dst)
    y3 = _layer(N, H, acc2, y2, dinv, b2, g2, be2, W3, True, True)
    acc3 = _make_msgpass(N, E, H)(y3, src, dst)
    x3 = _layer(N, H, acc3, y3, dinv, b3, g3, be3, None, False, False)

    sump, maxp = _make_pool(N, G, H)(x3, batch)

    return _make_tc_head(G, H, ADD)(
        sump, maxp, cntp, additional_features, M1, mb1, mg1, mbe1, M2, mb2,
        mg2, mbe2, M3, mb3)
